# Initial kernel scaffold; baseline (speedup 1.0000x reference)
import jax
import jax.numpy as jnp
from jax.experimental import pallas as pl


def kernel(x, edge_index, W1, b1, W2, b2, att_src, att_dst):
    raise NotImplementedError("write your pallas kernel here")


# speedup vs baseline: 25.4012x; 25.4012x over previous
import functools

import jax
import jax.numpy as jnp
from jax import lax
from jax.experimental import pallas as pl
from jax.experimental.pallas import tpu as pltpu, tpu_sc as plsc

N = 100000
E = 3200000
IN_DIM = 16
HID_DIM = 32
OUT_DIM = 16

NC = 2
NS = 16
L = 16

HALF = N // NC
TRASH = HALF
ACCP = 50048
SPAN = ACCP // NS

G = 64
EDGES_PER_TILE = E // NS

AL_PAD = 100096
AL_SPAN = AL_PAD // NS

_SC_PARAMS = dict(
    compiler_params=pltpu.CompilerParams(use_tc_tiling_on_sc=False),
)


def _mesh():
    return plsc.VectorSubcoreMesh(core_axis_name="c", subcore_axis_name="s")


def _pieces(span, maxsz):
    out = []
    while span:
        t = min(maxsz, span)
        out.append(t)
        span -= t
    return out


def _zero_vmem(ref, words):
    flat_rows = words // L
    z = jnp.zeros((L,), jnp.float32)
    if ref.ndim == 1:
        def body(i, _):
            ref[pl.ds(i * L, L)] = z
            return 0
    else:
        per_row = ref.shape[1] // L
        def body(i, _):
            ref[i // per_row, pl.ds((i % per_row) * L, L)] = z
            return 0
    lax.fori_loop(0, flat_rows, body, 0)


def _zero_span(acc, vbuf, base, maxsz):
    off = 0
    for sz in _pieces(SPAN, maxsz):
        pltpu.sync_copy(vbuf.at[pl.ds(0, sz)], acc.at[pl.ds(base + off, sz)])
        off += sz


def _copy_out(acc, hbm, vbuf, base_acc, base_hbm, maxsz):
    off = 0
    for sz in _pieces(SPAN, maxsz):
        pltpu.sync_copy(acc.at[pl.ds(base_acc + off, sz)], vbuf.at[pl.ds(0, sz)])
        pltpu.sync_copy(vbuf.at[pl.ds(0, sz)], hbm.at[pl.ds(base_hbm + off, sz)])
        off += sz


def _mask_chunk(d1v, idxv, lo, hi, rr, s1v=None, s2v=None):
    per_row = G // L
    def body(k, _):
        j = k // per_row
        o = (k % per_row) * L
        dv = d1v[pl.ds(k * L, L)]
        inr = (dv >= lo) & (dv < hi)
        idxv[j, pl.ds(o, L)] = jnp.where(inr, dv - lo, TRASH)
        if s1v is not None:
            s2v[j, pl.ds(o, L)] = s1v[pl.ds(k * L, L)]
        return 0
    lax.fori_loop(0, (rr * G) // L, body, 0)


R1 = 25
CH1 = R1 * G


def _deg_body(d_hbm, degp_hbm, dacc, d1v, idxv, ones, zv, sem):
    cid = lax.axis_index("c")
    sid = lax.axis_index("s")
    lo = cid * HALF
    hi = lo + HALF

    def fill_ones(i, _):
        ones[pl.ds(i * L, L)] = jnp.full((L,), 1.0, jnp.float32)
        return 0
    lax.fori_loop(0, G // L, fill_ones, 0)
    _zero_vmem(zv, CH1)
    _zero_span(dacc, zv, sid * SPAN, 1568)
    plsc.subcore_barrier()

    def chunk(it, _):
        b = sid * EDGES_PER_TILE + it * CH1
        pltpu.sync_copy(d_hbm.at[pl.ds(b, CH1)], d1v)
        _mask_chunk(d1v, idxv, lo, hi, R1)
        descs = [pltpu.async_copy(ones, dacc.at[idxv.at[j]], sem, add=True)
                 for j in range(R1)]
        for dsc in descs:
            dsc.wait()
        return 0
    lax.fori_loop(0, EDGES_PER_TILE // CH1, chunk, 0)

    plsc.subcore_barrier()
    _copy_out(dacc, degp_hbm, zv, sid * SPAN, cid * ACCP + sid * SPAN, 1568)


def _deg_call(d1):
    k = functools.partial(
        pl.kernel,
        out_type=jax.ShapeDtypeStruct((NC * ACCP,), jnp.float32),
        mesh=_mesh(),
        scratch_types=[
            pltpu.VMEM_SHARED((ACCP,), jnp.float32),
            pltpu.VMEM((CH1,), jnp.int32),
            pltpu.VMEM((R1, G), jnp.int32),
            pltpu.VMEM((G,), jnp.float32),
            pltpu.VMEM((CH1,), jnp.float32),
            pltpu.SemaphoreType.DMA,
        ],
        **_SC_PARAMS,
    )(_deg_body)
    return k(d1)


R2 = 5
CH2 = R2 * G


def _gcn_body(s_hbm, d_hbm, p_hbm, accp_hbm, acc, s1v, d1v, s2v, idxv, rows,
              gsem, ssem):
    cid = lax.axis_index("c")
    sid = lax.axis_index("s")
    lo = cid * HALF
    hi = lo + HALF

    _zero_vmem(rows, CH2 * HID_DIM)
    _zero_span(acc, rows, sid * SPAN, CH2)
    plsc.subcore_barrier()

    def chunk(it, _):
        b = sid * EDGES_PER_TILE + it * CH2
        pltpu.sync_copy(s_hbm.at[pl.ds(b, CH2)], s1v)
        pltpu.sync_copy(d_hbm.at[pl.ds(b, CH2)], d1v)
        _mask_chunk(d1v, idxv, lo, hi, R2, s1v, s2v)
        gd = [pltpu.async_copy(p_hbm.at[s2v.at[j]],
                               rows.at[pl.ds(j * G, G)], gsem)
              for j in range(R2)]
        for dsc in gd:
            dsc.wait()
        sd = [pltpu.async_copy(rows.at[pl.ds(j * G, G)],
                               acc.at[idxv.at[j]], ssem, add=True)
              for j in range(R2)]
        for dsc in sd:
            dsc.wait()
        return 0
    lax.fori_loop(0, EDGES_PER_TILE // CH2, chunk, 0)

    plsc.subcore_barrier()
    _copy_out(acc, accp_hbm, rows, sid * SPAN, cid * ACCP + sid * SPAN, CH2)


def _gcn_call(s1, d1, p):
    k = functools.partial(
        pl.kernel,
        out_type=jax.ShapeDtypeStruct((NC * ACCP, HID_DIM), jnp.float32),
        mesh=_mesh(),
        scratch_types=[
            pltpu.VMEM_SHARED((ACCP, HID_DIM), jnp.float32),
            pltpu.VMEM((CH2,), jnp.int32),
            pltpu.VMEM((CH2,), jnp.int32),
            pltpu.VMEM((R2, G), jnp.int32),
            pltpu.VMEM((R2, G), jnp.int32),
            pltpu.VMEM((CH2, HID_DIM), jnp.float32),
            pltpu.SemaphoreType.DMA,
            pltpu.SemaphoreType.DMA,
        ],
        **_SC_PARAMS,
    )(_gcn_body)
    return k(s1, d1, p)


R3 = 25
CH3 = R3 * G


def _gat_body(s_hbm, d_hbm, g_hbm, al_hbm, ar_hbm, cvec_hbm,
              acc2p_hbm, esump_hbm,
              acc2, esacc, alsp, arsp,
              s1v, d1v, s2v, idxv, rows, alv, arv, eev, cv, albuf,
              gsem, asem, ssem):
    cid = lax.axis_index("c")
    sid = lax.axis_index("s")
    lo = cid * HALF
    hi = lo + HALF

    pltpu.sync_copy(cvec_hbm, cv)
    pltpu.sync_copy(al_hbm.at[pl.ds(sid * AL_SPAN, AL_SPAN)], albuf)
    pltpu.sync_copy(albuf, alsp.at[pl.ds(sid * AL_SPAN, AL_SPAN)])
    pltpu.sync_copy(ar_hbm.at[pl.ds(cid * HALF + sid * SPAN, SPAN)],
                    albuf.at[pl.ds(0, SPAN)])
    pltpu.sync_copy(albuf.at[pl.ds(0, SPAN)], arsp.at[pl.ds(sid * SPAN, SPAN)])
    _zero_vmem(rows, CH3 * OUT_DIM)
    _zero_vmem(eev, CH3)
    _zero_span(acc2, rows, sid * SPAN, 1568)
    _zero_span(esacc, eev, sid * SPAN, 1568)
    plsc.subcore_barrier()

    cval = cv[...]

    def chunk(it, _):
        b = sid * EDGES_PER_TILE + it * CH3
        pltpu.sync_copy(s_hbm.at[pl.ds(b, CH3)], s1v)
        pltpu.sync_copy(d_hbm.at[pl.ds(b, CH3)], d1v)
        _mask_chunk(d1v, idxv, lo, hi, R3, s1v, s2v)
        ad = [pltpu.async_copy(alsp.at[s2v.at[j]],
                               alv.at[pl.ds(j * G, G)], asem)
              for j in range(R3)]
        rd = [pltpu.async_copy(arsp.at[idxv.at[j]],
                               arv.at[pl.ds(j * G, G)], asem)
              for j in range(R3)]
        gd = [pltpu.async_copy(g_hbm.at[s2v.at[j]],
                               rows.at[pl.ds(j * G, G)], gsem)
              for j in range(R3)]
        for dsc in ad + rd:
            dsc.wait()

        def ee_grp(k, _):
            e = alv[pl.ds(k * L, L)] + arv[pl.ds(k * L, L)]
            e = jnp.where(e >= 0.0, e, 0.2 * e)
            eev[pl.ds(k * L, L)] = jnp.exp(e - cval)
            return 0
        lax.fori_loop(0, CH3 // L, ee_grp, 0)

        for dsc in gd:
            dsc.wait()

        def scale_grp(k, _):
            base = k * L
            ee16 = eev[pl.ds(base, L)]
            for r in range(L):
                i = base + r
                rows[i, :] = rows[i, :] * ee16[r]
            return 0
        lax.fori_loop(0, CH3 // L, scale_grp, 0)

        sd = [pltpu.async_copy(rows.at[pl.ds(j * G, G)],
                               acc2.at[idxv.at[j]], ssem, add=True)
              for j in range(R3)]
        se = [pltpu.async_copy(eev.at[pl.ds(j * G, G)],
                               esacc.at[idxv.at[j]], ssem, add=True)
              for j in range(R3)]
        for dsc in sd + se:
            dsc.wait()
        return 0
    lax.fori_loop(0, EDGES_PER_TILE // CH3, chunk, 0)

    plsc.subcore_barrier()
    _copy_out(acc2, acc2p_hbm, rows, sid * SPAN, cid * ACCP + sid * SPAN, 1568)
    _copy_out(esacc, esump_hbm, eev, sid * SPAN, cid * ACCP + sid * SPAN, 1568)


def _gat_call(s1, d1, g, al_pad, ar_pad, cvec):
    k = functools.partial(
        pl.kernel,
        out_type=(jax.ShapeDtypeStruct((NC * ACCP, OUT_DIM), jnp.float32),
                  jax.ShapeDtypeStruct((NC * ACCP,), jnp.float32)),
        mesh=_mesh(),
        scratch_types=[
            pltpu.VMEM_SHARED((ACCP, OUT_DIM), jnp.float32),
            pltpu.VMEM_SHARED((ACCP,), jnp.float32),
            pltpu.VMEM_SHARED((AL_PAD,), jnp.float32),
            pltpu.VMEM_SHARED((ACCP,), jnp.float32),
            pltpu.VMEM((CH3,), jnp.int32),
            pltpu.VMEM((CH3,), jnp.int32),
            pltpu.VMEM((R3, G), jnp.int32),
            pltpu.VMEM((R3, G), jnp.int32),
            pltpu.VMEM((CH3, OUT_DIM), jnp.float32),
            pltpu.VMEM((CH3,), jnp.float32),
            pltpu.VMEM((CH3,), jnp.float32),
            pltpu.VMEM((CH3,), jnp.float32),
            pltpu.VMEM((L,), jnp.float32),
            pltpu.VMEM((AL_SPAN,), jnp.float32),
            pltpu.SemaphoreType.DMA,
            pltpu.SemaphoreType.DMA,
            pltpu.SemaphoreType.DMA,
        ],
        **_SC_PARAMS,
    )(_gat_body)
    return k(s1, d1, g, al_pad, ar_pad, cvec)


BT = 4000
GRID = N // BT


def _t1_body(deg_ref, x_ref, w1_ref, p_ref, dinv_ref):
    dinv = lax.rsqrt(deg_ref[...] + 1.0)
    h = jnp.dot(x_ref[...], w1_ref[...], preferred_element_type=jnp.float32)
    p_ref[...] = h * dinv
    dinv_ref[...] = dinv


def _t1_call(deg, x, W1):
    return pl.pallas_call(
        _t1_body,
        grid=(GRID,),
        in_specs=[
            pl.BlockSpec((BT, 1), lambda i: (i, 0)),
            pl.BlockSpec((BT, IN_DIM), lambda i: (i, 0)),
            pl.BlockSpec((IN_DIM, HID_DIM), lambda i: (0, 0)),
        ],
        out_specs=[
            pl.BlockSpec((BT, HID_DIM), lambda i: (i, 0)),
            pl.BlockSpec((BT, 1), lambda i: (i, 0)),
        ],
        out_shape=[
            jax.ShapeDtypeStruct((N, HID_DIM), jnp.float32),
            jax.ShapeDtypeStruct((N, 1), jnp.float32),
        ],
    )(deg, x, W1)


def _t2_body(acc1_ref, p_ref, dinv_ref, w2_ref, b1_ref, as_ref, ad_ref,
             g_ref, al_ref, ar_ref, mal_ref, mar_ref):
    out1 = dinv_ref[...] * (acc1_ref[...] + p_ref[...]) + b1_ref[...]
    xg = jnp.maximum(out1, 0.0)
    g = jnp.dot(xg, w2_ref[...], preferred_element_type=jnp.float32)
    g_ref[...] = g
    al = jnp.sum(g * as_ref[...], axis=1, keepdims=True)
    ar = jnp.sum(g * ad_ref[...], axis=1, keepdims=True)
    al_ref[...] = al
    ar_ref[...] = ar
    i = pl.program_id(0)

    @pl.when(i == 0)
    def _():
        mal_ref[...] = jnp.full((1, 1), -1e30, jnp.float32)
        mar_ref[...] = jnp.full((1, 1), -1e30, jnp.float32)

    mal_ref[...] = jnp.maximum(mal_ref[...], jnp.max(al))
    mar_ref[...] = jnp.maximum(mar_ref[...], jnp.max(ar))


def _t2_call(acc1, p, dinv, W2, b1, att_src, att_dst):
    return pl.pallas_call(
        _t2_body,
        grid=(GRID,),
        in_specs=[
            pl.BlockSpec((BT, HID_DIM), lambda i: (i, 0)),
            pl.BlockSpec((BT, HID_DIM), lambda i: (i, 0)),
            pl.BlockSpec((BT, 1), lambda i: (i, 0)),
            pl.BlockSpec((HID_DIM, OUT_DIM), lambda i: (0, 0)),
            pl.BlockSpec((1, HID_DIM), lambda i: (0, 0)),
            pl.BlockSpec((1, OUT_DIM), lambda i: (0, 0)),
            pl.BlockSpec((1, OUT_DIM), lambda i: (0, 0)),
        ],
        out_specs=[
            pl.BlockSpec((BT, OUT_DIM), lambda i: (i, 0)),
            pl.BlockSpec((BT, 1), lambda i: (i, 0)),
            pl.BlockSpec((BT, 1), lambda i: (i, 0)),
            pl.BlockSpec((1, 1), lambda i: (0, 0)),
            pl.BlockSpec((1, 1), lambda i: (0, 0)),
        ],
        out_shape=[
            jax.ShapeDtypeStruct((N, OUT_DIM), jnp.float32),
            jax.ShapeDtypeStruct((N, 1), jnp.float32),
            jax.ShapeDtypeStruct((N, 1), jnp.float32),
            jax.ShapeDtypeStruct((1, 1), jnp.float32),
            jax.ShapeDtypeStruct((1, 1), jnp.float32),
        ],
    )(acc1, p, dinv, W2, b1, att_src, att_dst)


def _t3_body(acc2_ref, esum_ref, g_ref, al_ref, ar_ref, c_ref, b2_ref, out_ref):
    es = al_ref[...] + ar_ref[...]
    es = jnp.where(es >= 0.0, es, 0.2 * es)
    ees = jnp.exp(es - c_ref[...])
    out_ref[...] = ((acc2_ref[...] + ees * g_ref[...])
                    / (esum_ref[...] + ees) + b2_ref[...])


def _t3_call(acc2, esum, g, al, ar, c, b2):
    return pl.pallas_call(
        _t3_body,
        grid=(GRID,),
        in_specs=[
            pl.BlockSpec((BT, OUT_DIM), lambda i: (i, 0)),
            pl.BlockSpec((BT, 1), lambda i: (i, 0)),
            pl.BlockSpec((BT, OUT_DIM), lambda i: (i, 0)),
            pl.BlockSpec((BT, 1), lambda i: (i, 0)),
            pl.BlockSpec((BT, 1), lambda i: (i, 0)),
            pl.BlockSpec((1, 1), lambda i: (0, 0)),
            pl.BlockSpec((1, OUT_DIM), lambda i: (0, 0)),
        ],
        out_specs=pl.BlockSpec((BT, OUT_DIM), lambda i: (i, 0)),
        out_shape=jax.ShapeDtypeStruct((N, OUT_DIM), jnp.float32),
    )(acc2, esum, g, al, ar, c, b2)


def _unpad1(a):
    return a.reshape(NC, ACCP)[:, :HALF].reshape(N)


def _unpad2(a, d):
    return a.reshape(NC, ACCP, d)[:, :HALF].reshape(N, d)


def kernel(x, edge_index, W1, b1, W2, b2, att_src, att_dst):
    s1 = edge_index[0]
    d1 = edge_index[1]

    degp = _deg_call(d1)
    deg = _unpad1(degp).reshape(N, 1)

    p, dinv = _t1_call(deg, x, W1)

    acc1 = _unpad2(_gcn_call(s1, d1, p), HID_DIM)

    g, al, ar, mal, mar = _t2_call(acc1, p, dinv, W2,
                                   b1.reshape(1, HID_DIM),
                                   att_src.reshape(1, OUT_DIM),
                                   att_dst.reshape(1, OUT_DIM))

    m = mal + mar
    c = jnp.where(m >= 0.0, m, 0.2 * m)
    cvec = jnp.broadcast_to(c.reshape(1), (L,))
    al_pad = jnp.pad(al.reshape(N), (0, AL_PAD - N))
    ar_pad = jnp.pad(ar.reshape(N), (0, AL_PAD - N))

    acc2p, esump = _gat_call(s1, d1, g, al_pad, ar_pad, cvec)
    acc2 = _unpad2(acc2p, OUT_DIM)
    esum = _unpad1(esump).reshape(N, 1)

    return _t3_call(acc2, esum, g, al, ar, c, b2.reshape(1, OUT_DIM))


# speedup vs baseline: 37.1525x; 1.4626x over previous
import functools

import jax
import jax.numpy as jnp
from jax import lax
from jax.experimental import pallas as pl
from jax.experimental.pallas import tpu as pltpu, tpu_sc as plsc

N = 100000
E = 3200000
IN_DIM = 16
HID_DIM = 32
OUT_DIM = 16
W32 = 32

NC = 2
NS = 16
L = 16

HALF = N // NC
TRASH = HALF
ACCP = 50048
SPAN = ACCP // NS

NPAD = 100096
NSPAN = NPAD // NS

G = 64
EDGES_PER_TILE = E // NS

_SC_PARAMS = dict(
    compiler_params=pltpu.CompilerParams(use_tc_tiling_on_sc=False,
                                         needs_layout_passes=False),
)


def _mesh():
    return plsc.VectorSubcoreMesh(core_axis_name="c", subcore_axis_name="s")


def _pieces(span, maxsz):
    out = []
    while span:
        t = min(maxsz, span)
        out.append(t)
        span -= t
    return out


def _zero_vmem(ref, words):
    flat_rows = words // L
    z = jnp.zeros((L,), jnp.float32)
    if ref.ndim == 1:
        def body(i, _):
            ref[pl.ds(i * L, L)] = z
            return 0
    else:
        per_row = ref.shape[1] // L
        def body(i, _):
            ref[i // per_row, pl.ds((i % per_row) * L, L)] = z
            return 0
    lax.fori_loop(0, flat_rows, body, 0)


def _zero_span(acc, vbuf, base, span, maxsz):
    off = 0
    for sz in _pieces(span, maxsz):
        pltpu.sync_copy(vbuf.at[pl.ds(0, sz)], acc.at[pl.ds(base + off, sz)])
        off += sz


def _copy_out(acc, hbm, vbuf, base_acc, base_hbm, span, maxsz):
    off = 0
    for sz in _pieces(span, maxsz):
        pltpu.sync_copy(acc.at[pl.ds(base_acc + off, sz)], vbuf.at[pl.ds(0, sz)])
        pltpu.sync_copy(vbuf.at[pl.ds(0, sz)], hbm.at[pl.ds(base_hbm + off, sz)])
        off += sz


def _mask_chunk(d1v, idxv, lo, hi, rr, s1v=None, s2v=None):
    per_row = G // L
    def body(k, _):
        j = k // per_row
        o = (k % per_row) * L
        dv = d1v[pl.ds(k * L, L)]
        inr = (dv >= lo) & (dv < hi)
        idxv[j, pl.ds(o, L)] = jnp.where(inr, dv - lo, TRASH)
        if s1v is not None:
            s2v[j, pl.ds(o, L)] = s1v[pl.ds(k * L, L)]
        return 0
    lax.fori_loop(0, (rr * G) // L, body, 0)


G1 = 32
R1 = 25
CH1 = R1 * G1
DROWS_PER_SC = (E // NC) // G1
DROWS_PER_TILE = DROWS_PER_SC // NS


def _deg_body(d2_hbm, degp_hbm, dacc, d2v, ones, zv, sem):
    cid = lax.axis_index("c")
    sid = lax.axis_index("s")

    def fill_ones(i, _):
        ones[pl.ds(i * L, L)] = jnp.full((L,), 1.0, jnp.float32)
        return 0
    lax.fori_loop(0, G1 // L, fill_ones, 0)
    _zero_vmem(zv, 1600)
    _zero_span(dacc, zv, sid * NSPAN, NSPAN, 1600)
    plsc.subcore_barrier()

    def chunk(it, _):
        b = cid * DROWS_PER_SC + sid * DROWS_PER_TILE + it * R1
        pltpu.sync_copy(d2_hbm.at[pl.ds(b, R1)], d2v)
        descs = [pltpu.async_copy(ones, dacc.at[d2v.at[j]], sem, add=True)
                 for j in range(R1)]
        for dsc in descs:
            dsc.wait()
        return 0
    lax.fori_loop(0, DROWS_PER_TILE // R1, chunk, 0)

    plsc.subcore_barrier()
    _copy_out(dacc, degp_hbm, zv, sid * NSPAN, cid * NPAD + sid * NSPAN,
              NSPAN, 1600)


def _deg_call(d2):
    k = functools.partial(
        pl.kernel,
        out_type=jax.ShapeDtypeStruct((NC * NPAD,), jnp.float32),
        mesh=_mesh(),
        scratch_types=[
            pltpu.VMEM_SHARED((NPAD,), jnp.float32),
            pltpu.VMEM((R1, G1), jnp.int32),
            pltpu.VMEM((G1,), jnp.float32),
            pltpu.VMEM((1600,), jnp.float32),
            pltpu.SemaphoreType.DMA,
        ],
        **_SC_PARAMS,
    )(_deg_body)
    return k(d2)


R2 = 5
CH2 = R2 * G


def _gcn_body(s_hbm, d_hbm, p_hbm, accp_hbm, acc, s1v, d1v, s2v, idxv, rows,
              gsem, ssem):
    cid = lax.axis_index("c")
    sid = lax.axis_index("s")
    lo = cid * HALF
    hi = lo + HALF

    _zero_vmem(rows, CH2 * HID_DIM)
    _zero_span(acc, rows, sid * SPAN, SPAN, CH2)
    plsc.subcore_barrier()

    def chunk(it, _):
        b = sid * EDGES_PER_TILE + it * CH2
        pltpu.sync_copy(s_hbm.at[pl.ds(b, CH2)], s1v)
        pltpu.sync_copy(d_hbm.at[pl.ds(b, CH2)], d1v)
        _mask_chunk(d1v, idxv, lo, hi, R2, s1v, s2v)
        gd = [pltpu.async_copy(p_hbm.at[s2v.at[j]],
                               rows.at[pl.ds(j * G, G)], gsem)
              for j in range(R2)]
        for dsc in gd:
            dsc.wait()
        sd = [pltpu.async_copy(rows.at[pl.ds(j * G, G)],
                               acc.at[idxv.at[j]], ssem, add=True)
              for j in range(R2)]
        for dsc in sd:
            dsc.wait()
        return 0
    lax.fori_loop(0, EDGES_PER_TILE // CH2, chunk, 0)

    plsc.subcore_barrier()
    _copy_out(acc, accp_hbm, rows, sid * SPAN, cid * ACCP + sid * SPAN,
              SPAN, CH2)


def _gcn_call(s1, d1, p):
    k = functools.partial(
        pl.kernel,
        out_type=jax.ShapeDtypeStruct((NC * ACCP, HID_DIM), jnp.float32),
        mesh=_mesh(),
        scratch_types=[
            pltpu.VMEM_SHARED((ACCP, HID_DIM), jnp.float32),
            pltpu.VMEM((CH2,), jnp.int32),
            pltpu.VMEM((CH2,), jnp.int32),
            pltpu.VMEM((R2, G), jnp.int32),
            pltpu.VMEM((R2, G), jnp.int32),
            pltpu.VMEM((CH2, HID_DIM), jnp.float32),
            pltpu.SemaphoreType.DMA,
            pltpu.SemaphoreType.DMA,
        ],
        **_SC_PARAMS,
    )(_gcn_body)
    return k(s1, d1, p)


R3 = 5
CH3 = R3 * G


def _gat_body(s_hbm, d_hbm, g32_hbm, ar_hbm, cvec_hbm, acc2p_hbm,
              acc2, arsp, s1v, d1v, s2v, idxv, grow, arv, cv,
              gsem, asem, ssem):
    cid = lax.axis_index("c")
    sid = lax.axis_index("s")
    lo = cid * HALF
    hi = lo + HALF

    pltpu.sync_copy(cvec_hbm, cv)
    off = 0
    for sz in _pieces(SPAN, CH3):
        pltpu.sync_copy(ar_hbm.at[pl.ds(cid * HALF + sid * SPAN + off, sz)],
                        arv.at[pl.ds(0, sz)])
        pltpu.sync_copy(arv.at[pl.ds(0, sz)], arsp.at[pl.ds(sid * SPAN + off, sz)])
        off += sz
    _zero_vmem(grow, CH3 * W32)
    _zero_span(acc2, grow, sid * SPAN, SPAN, CH3)
    plsc.subcore_barrier()

    cval = cv[...]
    col16 = jnp.full((L,), OUT_DIM, jnp.int32)

    def chunk(it, _):
        b = sid * EDGES_PER_TILE + it * CH3
        pltpu.sync_copy(s_hbm.at[pl.ds(b, CH3)], s1v)
        pltpu.sync_copy(d_hbm.at[pl.ds(b, CH3)], d1v)
        _mask_chunk(d1v, idxv, lo, hi, R3, s1v, s2v)
        rd = [pltpu.async_copy(arsp.at[idxv.at[j]],
                               arv.at[pl.ds(j * G, G)], asem)
              for j in range(R3)]
        gd = [pltpu.async_copy(g32_hbm.at[s2v.at[j]],
                               grow.at[pl.ds(j * G, G)], gsem)
              for j in range(R3)]
        for dsc in rd + gd:
            dsc.wait()

        def ee_grp(k, _):
            base = k * L
            riota = base + lax.iota(jnp.int32, L)
            al16 = plsc.load_gather(grow, [riota, col16])
            e = al16 + arv[pl.ds(base, L)]
            e = jnp.where(e >= 0.0, e, 0.2 * e)
            ee = jnp.exp(e - cval)
            plsc.store_scatter(grow, [riota, col16], ee)
            for r in range(L):
                i = base + r
                grow[i, pl.ds(0, OUT_DIM)] = grow[i, pl.ds(0, OUT_DIM)] * ee[r]
            return 0
        lax.fori_loop(0, CH3 // L, ee_grp, 0)

        sd = [pltpu.async_copy(grow.at[pl.ds(j * G, G)],
                               acc2.at[idxv.at[j]], ssem, add=True)
              for j in range(R3)]
        for dsc in sd:
            dsc.wait()
        return 0
    lax.fori_loop(0, EDGES_PER_TILE // CH3, chunk, 0)

    plsc.subcore_barrier()
    _copy_out(acc2, acc2p_hbm, grow, sid * SPAN, cid * ACCP + sid * SPAN,
              SPAN, CH3)


def _gat_call(s1, d1, g32, ar_pad, cvec):
    k = functools.partial(
        pl.kernel,
        out_type=jax.ShapeDtypeStruct((NC * ACCP, W32), jnp.float32),
        mesh=_mesh(),
        scratch_types=[
            pltpu.VMEM_SHARED((ACCP, W32), jnp.float32),
            pltpu.VMEM_SHARED((ACCP,), jnp.float32),
            pltpu.VMEM((CH3,), jnp.int32),
            pltpu.VMEM((CH3,), jnp.int32),
            pltpu.VMEM((R3, G), jnp.int32),
            pltpu.VMEM((R3, G), jnp.int32),
            pltpu.VMEM((CH3, W32), jnp.float32),
            pltpu.VMEM((CH3,), jnp.float32),
            pltpu.VMEM((L,), jnp.float32),
            pltpu.SemaphoreType.DMA,
            pltpu.SemaphoreType.DMA,
            pltpu.SemaphoreType.DMA,
        ],
        **_SC_PARAMS,
    )(_gat_body)
    return k(s1, d1, g32, ar_pad, cvec)


BT = 4000
GRID = N // BT


def _t1_body(d0_ref, d1_ref, x_ref, w1_ref, p_ref, dinv_ref):
    dinv = lax.rsqrt(d0_ref[...] + d1_ref[...] + 1.0)
    h = jnp.dot(x_ref[...], w1_ref[...], preferred_element_type=jnp.float32)
    p_ref[...] = h * dinv
    dinv_ref[...] = dinv


def _t1_call(d0, d1, x, W1):
    return pl.pallas_call(
        _t1_body,
        grid=(GRID,),
        in_specs=[
            pl.BlockSpec((BT, 1), lambda i: (i, 0)),
            pl.BlockSpec((BT, 1), lambda i: (i, 0)),
            pl.BlockSpec((BT, IN_DIM), lambda i: (i, 0)),
            pl.BlockSpec((IN_DIM, HID_DIM), lambda i: (0, 0)),
        ],
        out_specs=[
            pl.BlockSpec((BT, HID_DIM), lambda i: (i, 0)),
            pl.BlockSpec((BT, 1), lambda i: (i, 0)),
        ],
        out_shape=[
            jax.ShapeDtypeStruct((N, HID_DIM), jnp.float32),
            jax.ShapeDtypeStruct((N, 1), jnp.float32),
        ],
    )(d0, d1, x, W1)


def _t2_body(acc1_ref, p_ref, dinv_ref, w2_ref, b1_ref, as_ref, ad_ref,
             g32_ref, ar_ref, mal_ref, mar_ref):
    out1 = dinv_ref[...] * (acc1_ref[...] + p_ref[...]) + b1_ref[...]
    xg = jnp.maximum(out1, 0.0)
    g = jnp.dot(xg, w2_ref[...], preferred_element_type=jnp.float32)
    al = jnp.sum(g * as_ref[...], axis=1, keepdims=True)
    ar = jnp.sum(g * ad_ref[...], axis=1, keepdims=True)
    pad = jnp.zeros((g.shape[0], W32 - OUT_DIM - 1), jnp.float32)
    g32_ref[...] = jnp.concatenate([g, al, pad], axis=1)
    ar_ref[...] = ar
    i = pl.program_id(0)

    @pl.when(i == 0)
    def _():
        mal_ref[...] = jnp.full((1, 1), -1e30, jnp.float32)
        mar_ref[...] = jnp.full((1, 1), -1e30, jnp.float32)

    mal_ref[...] = jnp.maximum(mal_ref[...], jnp.max(al))
    mar_ref[...] = jnp.maximum(mar_ref[...], jnp.max(ar))


def _t2_call(acc1, p, dinv, W2, b1, att_src, att_dst):
    return pl.pallas_call(
        _t2_body,
        grid=(GRID,),
        in_specs=[
            pl.BlockSpec((BT, HID_DIM), lambda i: (i, 0)),
            pl.BlockSpec((BT, HID_DIM), lambda i: (i, 0)),
            pl.BlockSpec((BT, 1), lambda i: (i, 0)),
            pl.BlockSpec((HID_DIM, OUT_DIM), lambda i: (0, 0)),
            pl.BlockSpec((1, HID_DIM), lambda i: (0, 0)),
            pl.BlockSpec((1, OUT_DIM), lambda i: (0, 0)),
            pl.BlockSpec((1, OUT_DIM), lambda i: (0, 0)),
        ],
        out_specs=[
            pl.BlockSpec((BT, W32), lambda i: (i, 0)),
            pl.BlockSpec((BT, 1), lambda i: (i, 0)),
            pl.BlockSpec((1, 1), lambda i: (0, 0)),
            pl.BlockSpec((1, 1), lambda i: (0, 0)),
        ],
        out_shape=[
            jax.ShapeDtypeStruct((N, W32), jnp.float32),
            jax.ShapeDtypeStruct((N, 1), jnp.float32),
            jax.ShapeDtypeStruct((1, 1), jnp.float32),
            jax.ShapeDtypeStruct((1, 1), jnp.float32),
        ],
    )(acc1, p, dinv, W2, b1, att_src, att_dst)


def _t3_body(a32_ref, g32_ref, ar_ref, c_ref, b2_ref, out_ref):
    a32 = a32_ref[...]
    acc2 = a32[:, :OUT_DIM]
    esum = a32[:, OUT_DIM:]
    v17 = g32_ref[...]
    g = v17[:, :OUT_DIM]
    al = v17[:, OUT_DIM:]
    es = al + ar_ref[...]
    es = jnp.where(es >= 0.0, es, 0.2 * es)
    ees = jnp.exp(es - c_ref[...])
    out_ref[...] = (acc2 + ees * g) / (esum + ees) + b2_ref[...]


def _t3_call(a32, g32, ar, c, b2):
    return pl.pallas_call(
        _t3_body,
        grid=(GRID,),
        in_specs=[
            pl.BlockSpec((BT, W32), lambda i: (i, 0)),
            pl.BlockSpec((BT, W32), lambda i: (i, 0)),
            pl.BlockSpec((BT, 1), lambda i: (i, 0)),
            pl.BlockSpec((1, 1), lambda i: (0, 0)),
            pl.BlockSpec((1, OUT_DIM), lambda i: (0, 0)),
        ],
        out_specs=pl.BlockSpec((BT, OUT_DIM), lambda i: (i, 0)),
        out_shape=jax.ShapeDtypeStruct((N, OUT_DIM), jnp.float32),
    )(a32, g32, ar, c, b2)


def kernel(x, edge_index, W1, b1, W2, b2, att_src, att_dst):
    s1 = edge_index[0]
    d1 = edge_index[1]
    d2 = d1.reshape(E // G1, G1)

    degp = _deg_call(d2)
    d0 = degp[:N].reshape(N, 1)
    dd1 = degp[NPAD:NPAD + N].reshape(N, 1)

    p, dinv = _t1_call(d0, dd1, x, W1)

    acc1 = (_gcn_call(s1, d1, p)
            .reshape(NC, ACCP, HID_DIM)[:, :HALF].reshape(N, HID_DIM))

    g32, ar, mal, mar = _t2_call(acc1, p, dinv, W2,
                                 b1.reshape(1, HID_DIM),
                                 att_src.reshape(1, OUT_DIM),
                                 att_dst.reshape(1, OUT_DIM))

    m = mal + mar
    c = jnp.where(m >= 0.0, m, 0.2 * m)
    cvec = jnp.broadcast_to(c.reshape(1), (L,))
    ar_pad = jnp.pad(ar.reshape(N), (0, NPAD - N))

    a32 = (_gat_call(s1, d1, g32, ar_pad, cvec)
           .reshape(NC, ACCP, W32)[:, :HALF].reshape(N, W32))

    return _t3_call(a32, g32, ar, c, b2.reshape(1, OUT_DIM))
